# baseline (speedup 1.0000x reference)
import functools
import jax
import jax.numpy as jnp
from jax import lax
from jax.experimental import pallas as pl
from jax.experimental.pallas import tpu as pltpu, tpu_sc as plsc

SEQ = 8192
BATCH = 4
D = 1024
NW = 32
ROWS_PER_W = SEQ // NW
CHUNK = 4
NCHUNK = ROWS_PER_W // CHUNK
NBUF = 2
GROUPS = CHUNK * (D // 16)


def _sc_body(x_hbm, pos_hbm, out_hbm, xin, pin, xout, insems, outsems):
    c = lax.axis_index("c")
    s = lax.axis_index("s")
    wid = s * 2 + c
    base = wid * ROWS_PER_W

    def start_in(chunk, slot):
        row = base + chunk * CHUNK
        pltpu.async_copy(x_hbm.at[pl.ds(row, CHUNK)], xin.at[slot], insems.at[slot])
        pltpu.async_copy(pos_hbm.at[pl.ds(row, CHUNK)], pin.at[slot], insems.at[slot])

    def compute(slot):
        xb = xin.at[slot]
        pb = pin.at[slot]
        ob = xout.at[slot]

        def body(g, carry):
            sr = g // (D // 16)
            j = g % (D // 16)
            off = j * 16
            pvec = pb[sr, pl.ds(off, 16)]
            for b in range(BATCH):
                ob[sr, b, pl.ds(off, 16)] = xb[sr, b, pl.ds(off, 16)] + pvec
            return carry

        lax.fori_loop(0, GROUPS, body, 0, unroll=2)

    def loop(chunk, carry):
        slot = lax.rem(chunk, NBUF)
        pltpu.make_async_copy(x_hbm.at[pl.ds(0, CHUNK)], xin.at[slot], insems.at[slot]).wait()
        pltpu.make_async_copy(pos_hbm.at[pl.ds(0, CHUNK)], pin.at[slot], insems.at[slot]).wait()
        @pl.when(chunk >= NBUF)
        def drain_prev_out():
            pltpu.make_async_copy(xout.at[slot], out_hbm.at[pl.ds(0, CHUNK)], outsems.at[slot]).wait()

        compute(slot)
        row = base + chunk * CHUNK
        pltpu.async_copy(xout.at[slot], out_hbm.at[pl.ds(row, CHUNK)], outsems.at[slot])

        @pl.when(chunk + NBUF < NCHUNK)
        def refill():
            start_in(chunk + NBUF, slot)
        return carry

    for b in range(NBUF):
        start_in(b, b)
    lax.fori_loop(0, NCHUNK, loop, 0)
    for b in range(NBUF):
        slot = (NCHUNK - NBUF + b) % NBUF
        pltpu.make_async_copy(xout.at[slot], out_hbm.at[pl.ds(0, CHUNK)], outsems.at[slot]).wait()


def kernel(x, pos_embedding):
    mesh = plsc.VectorSubcoreMesh(core_axis_name="c", subcore_axis_name="s")
    k = functools.partial(
        pl.kernel,
        mesh=mesh,
        out_type=jax.ShapeDtypeStruct((SEQ, BATCH, D), jnp.float32),
        scratch_types=[
            pltpu.VMEM((NBUF, CHUNK, BATCH, D), jnp.float32),
            pltpu.VMEM((NBUF, CHUNK, D), jnp.float32),
            pltpu.VMEM((NBUF, CHUNK, BATCH, D), jnp.float32),
            pltpu.SemaphoreType.DMA((NBUF,)),
            pltpu.SemaphoreType.DMA((NBUF,)),
        ],
    )(_sc_body)
    return k(x, pos_embedding[:SEQ])


# speedup vs baseline: 3.8043x; 3.8043x over previous
import jax
import jax.numpy as jnp
from jax.experimental import pallas as pl


def _add_kernel(x_ref, pos_ref, o_ref):
    o_ref[...] = x_ref[...] + pos_ref[...][:, None, :]


def kernel(x, pos_embedding):
    seq, batch, d = x.shape
    tile = 512
    grid = (seq // tile,)
    return pl.pallas_call(
        _add_kernel,
        grid=grid,
        in_specs=[
            pl.BlockSpec((tile, batch, d), lambda i: (i, 0, 0)),
            pl.BlockSpec((tile, d), lambda i: (i, 0)),
        ],
        out_specs=pl.BlockSpec((tile, batch, d), lambda i: (i, 0, 0)),
        out_shape=jax.ShapeDtypeStruct((seq, batch, d), x.dtype),
    )(x, pos_embedding[:seq])
